# baseline (device time: 242698 ns/iter reference)
import functools
import math

import jax
import jax.numpy as jnp
from jax import lax
from jax.experimental import pallas as pl
from jax.experimental.pallas import tpu as pltpu

N_DEV = 32
B = 2
S_LOC = 128
S_GLB = N_DEV * S_LOC
D = 512
HQ = 4
DH = 64
HD = HQ * DH
R = B * S_LOC


def kernel(x, Wq, Wk, Wv, Wo):
    def body(x_ref, wq_ref, wk_ref, wv_ref, wo_ref, out_ref,
             kvg_ref, send_sems, recv_sems):
        me = lax.axis_index("i")

        x2 = x_ref[...].reshape(R, D)
        q2 = jnp.dot(x2, wq_ref[...], preferred_element_type=jnp.float32)
        k2 = jnp.dot(x2, wk_ref[...], preferred_element_type=jnp.float32)
        v2 = jnp.dot(x2, wv_ref[...], preferred_element_type=jnp.float32)

        row = lax.broadcasted_iota(jnp.int32, (R, HD), 0)
        col = lax.broadcasted_iota(jnp.int32, (R, HD), 1)
        s_loc = row % S_LOC
        pos = (me * S_LOC + s_loc).astype(jnp.float32)
        freq = ((col % DH) // 2).astype(jnp.float32)
        inv = jnp.exp(freq * (-2.0 * math.log(10000.0) / DH))
        ang = pos * inv
        cosm = jnp.cos(ang)
        sinm = jnp.sin(ang)
        even = (col % 2) == 0

        def rope(t):
            t_r = jnp.where(even, -jnp.roll(t, -1, axis=1),
                            jnp.roll(t, 1, axis=1))
            return t * cosm + t_r * sinm

        q2 = rope(q2)
        k2 = rope(k2)

        my_off = me * S_LOC
        kvg_ref[:, pl.ds(my_off, S_LOC), 0:HD] = k2.reshape(B, S_LOC, HD)
        kvg_ref[:, pl.ds(my_off, S_LOC), HD:2 * HD] = v2.reshape(B, S_LOC, HD)

        def kv_copy(chunk_pos, peer):
            return pltpu.make_async_remote_copy(
                src_ref=kvg_ref.at[:, pl.ds(chunk_pos * S_LOC, S_LOC), :],
                dst_ref=kvg_ref.at[:, pl.ds(chunk_pos * S_LOC, S_LOC), :],
                send_sem=send_sems.at[peer],
                recv_sem=recv_sems.at[chunk_pos],
                device_id=(peer,),
                device_id_type=pl.DeviceIdType.MESH,
            )

        for o in range(1, N_DEV):
            d = lax.rem(me + o, N_DEV)
            kv_copy(me, d).start()

        for o in range(1, N_DEV):
            p = lax.rem(me + o, N_DEV)
            kv_copy(p, p).wait_recv()

        ctx_rows = []
        for b in range(B):
            ctx_heads = []
            for h in range(HQ):
                q = q2[b * S_LOC:(b + 1) * S_LOC, h * DH:(h + 1) * DH]
                k = kvg_ref[b, :, h * DH:(h + 1) * DH]
                v = kvg_ref[b, :, HD + h * DH:HD + (h + 1) * DH]
                s = lax.dot_general(
                    q, k, (((1,), (1,)), ((), ())),
                    preferred_element_type=jnp.float32,
                ) * 0.125
                m = jnp.max(s, axis=1, keepdims=True)
                w = jnp.exp(s - m)
                w = w / jnp.sum(w, axis=1, keepdims=True)
                ctx_heads.append(
                    jnp.dot(w, v, preferred_element_type=jnp.float32))
            ctx_rows.append(jnp.concatenate(ctx_heads, axis=1))
        ctx2 = jnp.concatenate(ctx_rows, axis=0)

        out2 = jnp.dot(ctx2, wo_ref[...], preferred_element_type=jnp.float32)
        out_ref[...] = out2.reshape(B, S_LOC, D)

        for o in range(1, N_DEV):
            d = lax.rem(me + o, N_DEV)
            kv_copy(me, d).wait_send()

    return pl.pallas_call(
        body,
        out_shape=jax.ShapeDtypeStruct((B, S_LOC, D), jnp.float32),
        in_specs=[pl.BlockSpec(memory_space=pltpu.VMEM)] * 5,
        out_specs=pl.BlockSpec(memory_space=pltpu.VMEM),
        scratch_shapes=[
            pltpu.VMEM((B, S_GLB, 2 * HD), jnp.float32),
            pltpu.SemaphoreType.DMA((N_DEV,)),
            pltpu.SemaphoreType.DMA((N_DEV,)),
        ],
        compiler_params=pltpu.CompilerParams(
            vmem_limit_bytes=100 * 1024 * 1024,
        ),
    )(x, Wq, Wk, Wv, Wo)


# device time: 133655 ns/iter; 1.8159x vs baseline; 1.8159x over previous
import functools
import math

import jax
import jax.numpy as jnp
from jax import lax
from jax.experimental import pallas as pl
from jax.experimental.pallas import tpu as pltpu

N_DEV = 32
B = 2
S_LOC = 128
S_GLB = N_DEV * S_LOC
D = 512
HQ = 4
DH = 64
HD = HQ * DH
R = B * S_LOC


def kernel(x, Wq, Wk, Wv, Wo):
    def body(x_ref, wq_ref, wk_ref, wv_ref, wo_ref, out_ref,
             kvg_ref, send_sems, recv_sems):
        me = lax.axis_index("i")

        x2 = x_ref[...].reshape(R, D)
        q2 = jnp.dot(x2, wq_ref[...], preferred_element_type=jnp.float32)
        k2 = jnp.dot(x2, wk_ref[...], preferred_element_type=jnp.float32)
        v2 = jnp.dot(x2, wv_ref[...], preferred_element_type=jnp.float32)

        row = lax.broadcasted_iota(jnp.int32, (R, HD), 0)
        col = lax.broadcasted_iota(jnp.int32, (R, HD), 1)
        s_loc = row % S_LOC
        pos = (me * S_LOC + s_loc).astype(jnp.float32)
        freq = ((col % DH) // 2).astype(jnp.float32)
        inv = jnp.exp(freq * (-2.0 * math.log(10000.0) / DH))
        ang = pos * inv
        cosm = jnp.cos(ang)
        sinm = jnp.sin(ang)
        even = (col % 2) == 0

        def rope(t):
            t_r = jnp.where(even, -jnp.roll(t, -1, axis=1),
                            jnp.roll(t, 1, axis=1))
            return t * cosm + t_r * sinm

        q2 = rope(q2)
        k2 = rope(k2)

        my_off = me * S_LOC
        kvg_ref[:, pl.ds(my_off, S_LOC), 0:HD] = (
            k2.astype(jnp.bfloat16).reshape(B, S_LOC, HD))
        kvg_ref[:, pl.ds(my_off, S_LOC), HD:2 * HD] = (
            v2.astype(jnp.bfloat16).reshape(B, S_LOC, HD))

        def kv_copy(chunk_pos, peer):
            return pltpu.make_async_remote_copy(
                src_ref=kvg_ref.at[:, pl.ds(chunk_pos * S_LOC, S_LOC), :],
                dst_ref=kvg_ref.at[:, pl.ds(chunk_pos * S_LOC, S_LOC), :],
                send_sem=send_sems.at[peer],
                recv_sem=recv_sems.at[chunk_pos],
                device_id=(peer,),
                device_id_type=pl.DeviceIdType.MESH,
            )

        for o in range(1, N_DEV):
            d = lax.rem(me + o, N_DEV)
            kv_copy(me, d).start()

        for o in range(1, N_DEV):
            p = lax.rem(me + o, N_DEV)
            kv_copy(p, p).wait_recv()

        ctx_rows = []
        for b in range(B):
            ctx_heads = []
            for h in range(HQ):
                q = q2[b * S_LOC:(b + 1) * S_LOC, h * DH:(h + 1) * DH]
                q = q.astype(jnp.bfloat16)
                k = kvg_ref[b, :, h * DH:(h + 1) * DH]
                v = kvg_ref[b, :, HD + h * DH:HD + (h + 1) * DH]
                s = lax.dot_general(
                    q, k, (((1,), (1,)), ((), ())),
                    preferred_element_type=jnp.float32,
                ) * 0.125
                m = jnp.max(s, axis=1, keepdims=True)
                w = jnp.exp(s - m)
                w = (w / jnp.sum(w, axis=1, keepdims=True)).astype(jnp.bfloat16)
                ctx_heads.append(
                    jnp.dot(w, v, preferred_element_type=jnp.float32))
            ctx_rows.append(jnp.concatenate(ctx_heads, axis=1))
        ctx2 = jnp.concatenate(ctx_rows, axis=0)

        out2 = jnp.dot(ctx2, wo_ref[...], preferred_element_type=jnp.float32)
        out_ref[...] = out2.reshape(B, S_LOC, D)

        for o in range(1, N_DEV):
            d = lax.rem(me + o, N_DEV)
            kv_copy(me, d).wait_send()

    return pl.pallas_call(
        body,
        out_shape=jax.ShapeDtypeStruct((B, S_LOC, D), jnp.float32),
        in_specs=[pl.BlockSpec(memory_space=pltpu.VMEM)] * 5,
        out_specs=pl.BlockSpec(memory_space=pltpu.VMEM),
        scratch_shapes=[
            pltpu.VMEM((B, S_GLB, 2 * HD), jnp.bfloat16),
            pltpu.SemaphoreType.DMA((N_DEV,)),
            pltpu.SemaphoreType.DMA((N_DEV,)),
        ],
        compiler_params=pltpu.CompilerParams(
            vmem_limit_bytes=100 * 1024 * 1024,
        ),
    )(x, Wq, Wk, Wv, Wo)
